# Initial kernel scaffold; baseline (speedup 1.0000x reference)
import jax
import jax.numpy as jnp
from jax.experimental import pallas as pl


def kernel(features, edge_index, W1, b1, betas, Wc, bc):
    raise NotImplementedError("write your pallas kernel here")


# speedup vs baseline: 8.1075x; 8.1075x over previous
import functools

import jax
import jax.numpy as jnp
from jax import lax
from jax.experimental import pallas as pl
from jax.experimental.pallas import tpu as pltpu
from jax.experimental.pallas import tpu_sc as plsc

_L = 16


def _round_up(x, m):
    return (x + m - 1) // m * m


@functools.cache
def _build(N, E, D):
    f32 = jnp.float32
    i32 = jnp.int32

    NW = 32
    NSUB = 16
    RB = 1000
    assert N % RB == 0
    GRID = N // RB

    def _proj_body(x_ref, w_ref, b_ref, beta_ref, h_ref, hn_ref, hnb_ref):
        h = jnp.dot(x_ref[...], w_ref[...], preferred_element_type=f32)
        h = jnp.maximum(h + b_ref[...], 0.0)
        ss = jnp.sum(h * h, axis=1, keepdims=True)
        hn = h / jnp.maximum(jnp.sqrt(ss), 1e-12)
        h_ref[...] = h
        hn_ref[...] = hn
        hnb_ref[...] = hn * beta_ref[0, 0]

    proj = pl.pallas_call(
        _proj_body,
        grid=(GRID,),
        in_specs=[
            pl.BlockSpec((RB, D), lambda i: (i, 0)),
            pl.BlockSpec((D, D), lambda i: (0, 0)),
            pl.BlockSpec((1, D), lambda i: (0, 0)),
            pl.BlockSpec((1, 1), lambda i: (0, 0)),
        ],
        out_specs=[pl.BlockSpec((RB, D), lambda i: (i, 0))] * 3,
        out_shape=[jax.ShapeDtypeStruct((N, D), f32)] * 3,
    )

    NP = _round_up(N, NSUB * _L)
    STRIPE = NP // NSUB

    def _sumnorm_body(p_ref, beta_ref, h_ref, hn_ref, hnb_ref):
        h = p_ref[0] + p_ref[1]
        ss = jnp.sum(h * h, axis=1, keepdims=True)
        hn = h / jnp.maximum(jnp.sqrt(ss), 1e-12)
        h_ref[...] = h
        hn_ref[...] = hn
        hnb_ref[...] = hn * beta_ref[0, 0]

    sumnorm = pl.pallas_call(
        _sumnorm_body,
        grid=(GRID,),
        in_specs=[
            pl.BlockSpec((2, RB, D), lambda i: (0, i, 0)),
            pl.BlockSpec((1, 1), lambda i: (0, 0)),
        ],
        out_specs=[pl.BlockSpec((RB, D), lambda i: (i, 0))] * 3,
        out_shape=[jax.ShapeDtypeStruct((N, D), f32)] * 3,
    )

    def _cls_body(p_ref, w_ref, b_ref, y_ref):
        h = p_ref[0] + p_ref[1]
        y_ref[...] = jnp.dot(h, w_ref[...], preferred_element_type=f32) + b_ref[...]

    cls = pl.pallas_call(
        _cls_body,
        grid=(GRID,),
        in_specs=[
            pl.BlockSpec((2, RB, D), lambda i: (0, i, 0)),
            pl.BlockSpec((D, D), lambda i: (0, 0)),
            pl.BlockSpec((1, D), lambda i: (0, 0)),
        ],
        out_specs=pl.BlockSpec((RB, D), lambda i: (i, 0)),
        out_shape=jax.ShapeDtypeStruct((N, D), f32),
    )

    B = 80
    EPW = E // NW
    assert EPW * NW == E and EPW % B == 0
    CH = EPW // B
    mesh = plsc.VectorSubcoreMesh(core_axis_name="c", subcore_axis_name="s")
    sc_params = pltpu.CompilerParams(needs_layout_passes=False)

    @functools.partial(
        pl.kernel, mesh=mesh,
        out_type=(
            jax.ShapeDtypeStruct((E,), f32),
            jax.ShapeDtypeStruct((2, NP), f32),
        ),
        scratch_types=[
            pltpu.VMEM((B,), i32),
            pltpu.VMEM((B,), i32),
            pltpu.VMEM((B, D), f32),
            pltpu.VMEM((B, D), f32),
            pltpu.VMEM((B,), f32),
            pltpu.VMEM_SHARED((NP,), f32),
            pltpu.SemaphoreType.DMA,
            pltpu.SemaphoreType.DMA,
        ],
        compiler_params=sc_params,
    )
    def pass_a(hn_hbm, hnb_hbm, src_hbm, dst_hbm, zvec_hbm, ez_hbm, den_hbm,
               idx_s, idx_d, rows_s, rows_d, ezb, den_sh, sem1, sem2):
        c = lax.axis_index("c")
        s = lax.axis_index("s")
        base = (c * NSUB + s) * EPW
        pltpu.sync_copy(zvec_hbm.at[pl.ds(s * STRIPE, STRIPE)],
                        den_sh.at[pl.ds(s * STRIPE, STRIPE)])
        plsc.subcore_barrier()

        def chunk(ci, carry):
            off = base + ci * B
            pltpu.sync_copy(src_hbm.at[pl.ds(off, B)], idx_s)
            pltpu.sync_copy(dst_hbm.at[pl.ds(off, B)], idx_d)
            cp1 = pltpu.async_copy(hnb_hbm.at[idx_s], rows_s, sem1)
            cp2 = pltpu.async_copy(hn_hbm.at[idx_d], rows_d, sem2)
            cp1.wait()
            cp2.wait()

            lane = lax.iota(jnp.int32, _L)
            perms = [lane ^ sh for sh in (8, 4, 2, 1)]

            def grp(g, cc):
                def edge16(j, tot):
                    e = g * _L + j
                    acc = rows_s[e, pl.ds(0, _L)] * rows_d[e, pl.ds(0, _L)]
                    for k in range(1, D // _L):
                        acc = acc + (rows_s[e, pl.ds(k * _L, _L)]
                                     * rows_d[e, pl.ds(k * _L, _L)])
                    for p in perms:
                        acc = acc + acc.at[p].get(mode="promise_in_bounds")
                    return jnp.where(lane == j, acc, tot)

                tot = lax.fori_loop(0, _L, edge16, jnp.zeros((_L,), f32))
                ezb[pl.ds(g * _L, _L)] = jnp.exp(tot)
                return cc

            lax.fori_loop(0, B // _L, grp, 0)
            pltpu.sync_copy(ezb, ez_hbm.at[pl.ds(off, B)])
            pltpu.sync_copy(ezb, den_sh.at[idx_d], add=True)
            return carry

        lax.fori_loop(0, CH, chunk, 0)
        plsc.subcore_barrier()
        pltpu.sync_copy(den_sh.at[pl.ds(s * STRIPE, STRIPE)],
                        den_hbm.at[c, pl.ds(s * STRIPE, STRIPE)])

    @functools.partial(
        pl.kernel, mesh=mesh,
        out_type=jax.ShapeDtypeStruct((2, NP, D), f32),
        scratch_types=[
            pltpu.VMEM((B,), i32),
            pltpu.VMEM((B,), i32),
            pltpu.VMEM((B, D), f32),
            pltpu.VMEM((B,), f32),
            pltpu.VMEM((NP,), f32),
            pltpu.VMEM((NP,), f32),
            pltpu.VMEM_SHARED((NP, D), f32),
            pltpu.SemaphoreType.DMA,
        ],
        compiler_params=sc_params,
    )
    def pass_b(h_hbm, ez_hbm, src_hbm, dst_hbm, den_hbm, zmat_hbm, out_hbm,
               idx_s, idx_d, rows, ezb, den_v, den_v2, out_sh, sem):
        c = lax.axis_index("c")
        s = lax.axis_index("s")
        base = (c * NSUB + s) * EPW
        pltpu.sync_copy(zmat_hbm.at[pl.ds(s * STRIPE, STRIPE)],
                        out_sh.at[pl.ds(s * STRIPE, STRIPE)])
        pltpu.sync_copy(den_hbm.at[0], den_v)
        pltpu.sync_copy(den_hbm.at[1], den_v2)

        def dsum(i, cc):
            den_v[pl.ds(i * _L, _L)] = (den_v[pl.ds(i * _L, _L)]
                                        + den_v2[pl.ds(i * _L, _L)])
            return cc

        lax.fori_loop(0, NP // _L, dsum, 0)
        plsc.subcore_barrier()

        def chunk(ci, carry):
            off = base + ci * B
            pltpu.sync_copy(src_hbm.at[pl.ds(off, B)], idx_s)
            pltpu.sync_copy(dst_hbm.at[pl.ds(off, B)], idx_d)
            pltpu.sync_copy(ez_hbm.at[pl.ds(off, B)], ezb)
            pltpu.async_copy(h_hbm.at[idx_s], rows, sem).wait()

            def grp(g, cc):
                d16 = idx_d[pl.ds(g * _L, _L)]
                den = plsc.load_gather(den_v, [d16])
                a16 = ezb[pl.ds(g * _L, _L)] / den
                for j in range(_L):
                    av = a16.at[jnp.full((_L,), j, jnp.int32)].get(
                        mode="promise_in_bounds")
                    e = g * _L + j
                    for k in range(D // _L):
                        rows[e, pl.ds(k * _L, _L)] = (
                            rows[e, pl.ds(k * _L, _L)] * av)
                return cc

            lax.fori_loop(0, B // _L, grp, 0)
            pltpu.sync_copy(rows, out_sh.at[idx_d], add=True)
            return carry

        lax.fori_loop(0, CH, chunk, 0)
        plsc.subcore_barrier()
        pltpu.sync_copy(out_sh.at[pl.ds(s * STRIPE, STRIPE)],
                        out_hbm.at[c, pl.ds(s * STRIPE, STRIPE)])

    return proj, sumnorm, cls, pass_a, pass_b, NP


def kernel(features, edge_index, W1, b1, betas, Wc, bc):
    N, D = features.shape
    E = edge_index.shape[1]
    proj, sumnorm, cls, pass_a, pass_b, NP = _build(N, E, D)

    src = edge_index[0]
    dst = edge_index[1]
    zvec = jnp.zeros((NP,), jnp.float32)
    zmat = jnp.zeros((NP, D), jnp.float32)

    h, hn, hnb = proj(features, W1, b1.reshape(1, D), betas[0].reshape(1, 1))
    for i in range(betas.shape[0]):
        ez, den = pass_a(hn, hnb, src, dst, zvec)
        part = pass_b(h, ez, src, dst, den, zmat)
        if i + 1 < betas.shape[0]:
            h, hn, hnb = sumnorm(part, betas[i + 1].reshape(1, 1))
    return cls(part, Wc, bc.reshape(1, D))


# speedup vs baseline: 9.0585x; 1.1173x over previous
import functools

import jax
import jax.numpy as jnp
from jax import lax
from jax.experimental import pallas as pl
from jax.experimental.pallas import tpu as pltpu
from jax.experimental.pallas import tpu_sc as plsc

_L = 16


def _round_up(x, m):
    return (x + m - 1) // m * m


@functools.cache
def _build(N, E, D):
    f32 = jnp.float32
    i32 = jnp.int32

    NW = 32
    NSUB = 16
    Dh = D // 2
    RB = 1000
    assert N % RB == 0 and D % (2 * _L) == 0
    GRID = N // RB

    NP = _round_up(N, NSUB * _L)
    STRIPE = NP // NSUB

    def _proj_body(x_ref, w_ref, b_ref, beta_ref, h2_ref, hn_ref, hnb_ref):
        h = jnp.dot(x_ref[...], w_ref[...], preferred_element_type=f32)
        h = jnp.maximum(h + b_ref[...], 0.0)
        ss = jnp.sum(h * h, axis=1, keepdims=True)
        hn = h / jnp.maximum(jnp.sqrt(ss), 1e-12)
        h2_ref[0] = h[:, :Dh]
        h2_ref[1] = h[:, Dh:]
        hn_ref[...] = hn
        hnb_ref[...] = hn * beta_ref[0, 0]

    proj = pl.pallas_call(
        _proj_body,
        grid=(GRID,),
        in_specs=[
            pl.BlockSpec((RB, D), lambda i: (i, 0)),
            pl.BlockSpec((D, D), lambda i: (0, 0)),
            pl.BlockSpec((1, D), lambda i: (0, 0)),
            pl.BlockSpec((1, 1), lambda i: (0, 0)),
        ],
        out_specs=[
            pl.BlockSpec((2, RB, Dh), lambda i: (0, i, 0)),
            pl.BlockSpec((RB, D), lambda i: (i, 0)),
            pl.BlockSpec((RB, D), lambda i: (i, 0)),
        ],
        out_shape=[
            jax.ShapeDtypeStruct((2, NP, Dh), f32),
            jax.ShapeDtypeStruct((N, D), f32),
            jax.ShapeDtypeStruct((N, D), f32),
        ],
    )

    def _norm_body(p_ref, beta_ref, hn_ref, hnb_ref):
        h = jnp.concatenate([p_ref[0], p_ref[1]], axis=1)
        ss = jnp.sum(h * h, axis=1, keepdims=True)
        hn = h / jnp.maximum(jnp.sqrt(ss), 1e-12)
        hn_ref[...] = hn
        hnb_ref[...] = hn * beta_ref[0, 0]

    norm = pl.pallas_call(
        _norm_body,
        grid=(GRID,),
        in_specs=[
            pl.BlockSpec((2, RB, Dh), lambda i: (0, i, 0)),
            pl.BlockSpec((1, 1), lambda i: (0, 0)),
        ],
        out_specs=[pl.BlockSpec((RB, D), lambda i: (i, 0))] * 2,
        out_shape=[jax.ShapeDtypeStruct((N, D), f32)] * 2,
    )

    def _cls_body(p_ref, w_ref, b_ref, y_ref):
        h = jnp.concatenate([p_ref[0], p_ref[1]], axis=1)
        y_ref[...] = jnp.dot(h, w_ref[...], preferred_element_type=f32) + b_ref[...]

    cls = pl.pallas_call(
        _cls_body,
        grid=(GRID,),
        in_specs=[
            pl.BlockSpec((2, RB, Dh), lambda i: (0, i, 0)),
            pl.BlockSpec((D, D), lambda i: (0, 0)),
            pl.BlockSpec((1, D), lambda i: (0, 0)),
        ],
        out_specs=pl.BlockSpec((RB, D), lambda i: (i, 0)),
        out_shape=jax.ShapeDtypeStruct((N, D), f32),
    )

    B = 80
    EPW = E // NW
    assert EPW * NW == E and EPW % B == 0 and B % _L == 0
    CH = EPW // B
    assert CH % 2 == 1
    EPT = E // NSUB
    CHB = EPT // B
    assert CHB % 2 == 0 and CHB >= 4
    mesh = plsc.VectorSubcoreMesh(core_axis_name="c", subcore_axis_name="s")
    sc_params = pltpu.CompilerParams(needs_layout_passes=False)
    sc_params_untiled = pltpu.CompilerParams(
        needs_layout_passes=False, use_tc_tiling_on_sc=False)

    @functools.partial(
        pl.kernel, mesh=mesh,
        out_type=(
            jax.ShapeDtypeStruct((NW, CH, B), f32),
            jax.ShapeDtypeStruct((2, NP), f32),
        ),
        scratch_types=[
            pltpu.VMEM((CH, B), i32),
            pltpu.VMEM((CH, B), i32),
            pltpu.VMEM((CH, B), f32),
            pltpu.VMEM((2, B, D), f32),
            pltpu.VMEM((2, B, D), f32),
            pltpu.VMEM_SHARED((NP,), f32),
            pltpu.SemaphoreType.DMA,
            pltpu.SemaphoreType.DMA,
            pltpu.SemaphoreType.DMA,
            pltpu.SemaphoreType.DMA,
        ],
        compiler_params=sc_params,
    )
    def pass_a(hn_hbm, hnb_hbm, src_hbm, dst_hbm, zvec_hbm, ez_hbm, den_hbm,
               idx_s, idx_d, ezv, rows_s, rows_d, den_sh, s0, s1, d0, d1):
        c = lax.axis_index("c")
        s = lax.axis_index("s")
        wid = c * NSUB + s
        pltpu.sync_copy(src_hbm.at[wid], idx_s)
        pltpu.sync_copy(dst_hbm.at[wid], idx_d)
        pltpu.sync_copy(zvec_hbm.at[pl.ds(s * STRIPE, STRIPE)],
                        den_sh.at[pl.ds(s * STRIPE, STRIPE)])
        plsc.subcore_barrier()

        sems = ((s0, d0), (s1, d1))

        def issue(ci, buf):
            ss_, dd_ = sems[buf]
            pltpu.async_copy(hnb_hbm.at[idx_s.at[ci]], rows_s.at[buf], ss_)
            pltpu.async_copy(hn_hbm.at[idx_d.at[ci]], rows_d.at[buf], dd_)

        def wait(buf):
            ss_, dd_ = sems[buf]
            pltpu.make_async_copy(hnb_hbm.at[idx_s.at[0]],
                                  rows_s.at[buf], ss_).wait()
            pltpu.make_async_copy(hn_hbm.at[idx_d.at[0]],
                                  rows_d.at[buf], dd_).wait()

        lane = lax.iota(i32, _L)
        perms = [lane ^ sh for sh in (8, 4, 2, 1)]

        def compute(ci, buf):
            def grpf(g, cc):
                tot = jnp.zeros((_L,), f32)
                for j in range(_L):
                    e = g * _L + j
                    acc = (rows_s[buf, e, pl.ds(0, _L)]
                           * rows_d[buf, e, pl.ds(0, _L)])
                    for k in range(1, D // _L):
                        acc = acc + (rows_s[buf, e, pl.ds(k * _L, _L)]
                                     * rows_d[buf, e, pl.ds(k * _L, _L)])
                    for p in perms:
                        acc = acc + acc.at[p].get(mode="promise_in_bounds")
                    tot = jnp.where(lane == j, acc, tot)
                ezv[ci, pl.ds(g * _L, _L)] = jnp.exp(tot)
                return cc

            lax.fori_loop(0, B // _L, grpf, 0)
            pltpu.sync_copy(ezv.at[ci], den_sh.at[idx_d.at[ci]], add=True)

        issue(0, 0)

        def body(g, cc):
            c0 = 2 * g
            issue(c0 + 1, 1)
            wait(0)
            compute(c0, 0)
            issue(c0 + 2, 0)
            wait(1)
            compute(c0 + 1, 1)
            return cc

        lax.fori_loop(0, (CH - 1) // 2, body, 0)
        wait(0)
        compute(CH - 1, 0)

        pltpu.sync_copy(ezv, ez_hbm.at[wid])
        plsc.subcore_barrier()
        pltpu.sync_copy(den_sh.at[pl.ds(s * STRIPE, STRIPE)],
                        den_hbm.at[c, pl.ds(s * STRIPE, STRIPE)])

    @functools.partial(
        pl.kernel, mesh=mesh,
        out_type=jax.ShapeDtypeStruct((2, NP, Dh), f32),
        scratch_types=[
            pltpu.VMEM((CHB, B), i32),
            pltpu.VMEM((CHB, B), i32),
            pltpu.VMEM((2, B), f32),
            pltpu.VMEM((2, B, Dh), f32),
            pltpu.VMEM((2, B, Dh), f32),
            pltpu.VMEM((NP,), f32),
            pltpu.VMEM((NP,), f32),
            pltpu.VMEM_SHARED((NP, Dh), f32),
            pltpu.SemaphoreType.DMA,
            pltpu.SemaphoreType.DMA,
            pltpu.SemaphoreType.DMA,
            pltpu.SemaphoreType.DMA,
            pltpu.SemaphoreType.DMA,
            pltpu.SemaphoreType.DMA,
        ],
        compiler_params=sc_params_untiled,
    )
    def pass_b(h2_hbm, ez_hbm, src_hbm, dst_hbm, den_hbm, zmat_hbm, out_hbm,
               idx_s, idx_d, ezb, rows, wrow, den_v, den_v2, out_sh,
               g0, g1, e0, e1, w0, w1):
        c = lax.axis_index("c")
        s = lax.axis_index("s")
        ht = h2_hbm.at[c]
        pltpu.sync_copy(src_hbm.at[s], idx_s)
        pltpu.sync_copy(dst_hbm.at[s], idx_d)
        pltpu.sync_copy(zmat_hbm.at[pl.ds(s * STRIPE, STRIPE)],
                        out_sh.at[pl.ds(s * STRIPE, STRIPE)])
        pltpu.sync_copy(den_hbm.at[0], den_v)
        pltpu.sync_copy(den_hbm.at[1], den_v2)

        def dsum(i, cc):
            den_v[pl.ds(i * _L, _L)] = (den_v[pl.ds(i * _L, _L)]
                                        + den_v2[pl.ds(i * _L, _L)])
            return cc

        lax.fori_loop(0, NP // _L, dsum, 0)
        plsc.subcore_barrier()

        gsems = (g0, g1)
        esems = (e0, e1)
        wsems = (w0, w1)

        def issue(ci, buf):
            pltpu.async_copy(ht.at[idx_s.at[ci]], rows.at[buf], gsems[buf])
            pltpu.async_copy(ez_hbm.at[s, ci], ezb.at[buf], esems[buf])

        def wait_g(buf):
            pltpu.make_async_copy(ht.at[idx_s.at[0]],
                                  rows.at[buf], gsems[buf]).wait()
            pltpu.make_async_copy(ez_hbm.at[s, 0],
                                  ezb.at[buf], esems[buf]).wait()

        def wait_w(buf):
            pltpu.make_async_copy(wrow.at[buf],
                                  out_sh.at[idx_d.at[0]], wsems[buf]).wait()

        def compute(ci, buf, prev_scatter):
            wait_g(buf)
            if prev_scatter is None:
                wait_w(buf)
            else:
                @pl.when(prev_scatter)
                def _():
                    wait_w(buf)

            def grpf(g2, cc):
                d16 = idx_d[ci, pl.ds(g2 * _L, _L)]
                den = plsc.load_gather(den_v, [d16])
                a16 = ezb[buf, pl.ds(g2 * _L, _L)] / den
                for j in range(_L):
                    av = a16.at[jnp.full((_L,), j, i32)].get(
                        mode="promise_in_bounds")
                    e = g2 * _L + j
                    for k in range(Dh // _L):
                        wrow[buf, e, pl.ds(k * _L, _L)] = (
                            rows[buf, e, pl.ds(k * _L, _L)] * av)
                return cc

            lax.fori_loop(0, B // _L, grpf, 0)
            pltpu.async_copy(wrow.at[buf], out_sh.at[idx_d.at[ci]],
                             wsems[buf], add=True)

        issue(0, 0)

        def body(g, cc):
            c0 = 2 * g
            issue(c0 + 1, 1)
            compute(c0, 0, g > 0)
            issue(c0 + 2, 0)
            compute(c0 + 1, 1, g > 0)
            return cc

        lax.fori_loop(0, CHB // 2 - 1, body, 0)
        issue(CHB - 1, 1)
        compute(CHB - 2, 0, None)
        compute(CHB - 1, 1, None)
        wait_w(0)
        wait_w(1)

        plsc.subcore_barrier()
        pltpu.sync_copy(out_sh.at[pl.ds(s * STRIPE, STRIPE)],
                        out_hbm.at[c, pl.ds(s * STRIPE, STRIPE)])

    return proj, norm, cls, pass_a, pass_b, NW, NSUB, CH, CHB, B


def kernel(features, edge_index, W1, b1, betas, Wc, bc):
    N, D = features.shape
    E = edge_index.shape[1]
    proj, norm, cls, pass_a, pass_b, NW, NSUB, CH, CHB, B = _build(N, E, D)
    NP = _round_up(N, NSUB * _L)

    src_a = edge_index[0].reshape(NW, CH, B)
    dst_a = edge_index[1].reshape(NW, CH, B)
    src_b = edge_index[0].reshape(NSUB, CHB, B)
    dst_b = edge_index[1].reshape(NSUB, CHB, B)
    zvec = jnp.zeros((NP,), jnp.float32)
    zmat = jnp.zeros((NP, D // 2), jnp.float32)

    part, hn, hnb = proj(features, W1, b1.reshape(1, D),
                         betas[0].reshape(1, 1))
    for i in range(betas.shape[0]):
        ez, den = pass_a(hn, hnb, src_a, dst_a, zvec)
        ezb = ez.reshape(NSUB, CHB, B)
        part = pass_b(part, ezb, src_b, dst_b, den, zmat)
        if i + 1 < betas.shape[0]:
            hn, hnb = norm(part, betas[i + 1].reshape(1, 1))
    return cls(part, Wc, bc.reshape(1, D))


# speedup vs baseline: 15.6767x; 1.7306x over previous
import functools

import jax
import jax.numpy as jnp
from jax import lax
from jax.experimental import pallas as pl
from jax.experimental.pallas import tpu as pltpu
from jax.experimental.pallas import tpu_sc as plsc

_L = 16


def _round_up(x, m):
    return (x + m - 1) // m * m


@functools.cache
def _build(N, E, D):
    f32 = jnp.float32
    i32 = jnp.int32

    NW = 32
    NSUB = 16
    Dh = D // 2
    RB = 1000
    assert N % RB == 0 and D % (2 * _L) == 0
    GRID = N // RB

    NP = _round_up(N, NSUB * _L)
    STRIPE = NP // NSUB

    Dp = D // 2
    Dhp = Dh // 2

    def _pack_bf16(x, rows):
        del rows
        m = x.shape[1] // 2
        u = lax.bitcast_convert_type(x, jnp.uint32)
        r = u + (((u >> 16) & 1) + 0x7FFF)
        r = r & jnp.uint32(0xFFFF0000)
        w = (r[:, :m] >> 16) | r[:, m:]
        return lax.bitcast_convert_type(w, f32)

    def _proj_body(x_ref, w_ref, b_ref, beta_ref, h2_ref, hn_ref, hnb_ref):
        h = jnp.dot(x_ref[...], w_ref[...], preferred_element_type=f32)
        h = jnp.maximum(h + b_ref[...], 0.0)
        ss = jnp.sum(h * h, axis=1, keepdims=True)
        hn = h / jnp.maximum(jnp.sqrt(ss), 1e-12)
        h2_ref[0] = _pack_bf16(h[:, :Dh], RB)
        h2_ref[1] = _pack_bf16(h[:, Dh:], RB)
        hn_ref[...] = _pack_bf16(hn, RB)
        hnb_ref[...] = _pack_bf16(hn * beta_ref[0, 0], RB)

    proj = pl.pallas_call(
        _proj_body,
        grid=(GRID,),
        in_specs=[
            pl.BlockSpec((RB, D), lambda i: (i, 0)),
            pl.BlockSpec((D, D), lambda i: (0, 0)),
            pl.BlockSpec((1, D), lambda i: (0, 0)),
            pl.BlockSpec((1, 1), lambda i: (0, 0)),
        ],
        out_specs=[
            pl.BlockSpec((2, RB, Dhp), lambda i: (0, i, 0)),
            pl.BlockSpec((RB, Dp), lambda i: (i, 0)),
            pl.BlockSpec((RB, Dp), lambda i: (i, 0)),
        ],
        out_shape=[
            jax.ShapeDtypeStruct((2, NP, Dhp), f32),
            jax.ShapeDtypeStruct((N, Dp), f32),
            jax.ShapeDtypeStruct((N, Dp), f32),
        ],
    )

    def _norm_body(p_ref, beta_ref, h2_ref, hn_ref, hnb_ref):
        h = jnp.concatenate([p_ref[0], p_ref[1]], axis=1)
        ss = jnp.sum(h * h, axis=1, keepdims=True)
        hn = h / jnp.maximum(jnp.sqrt(ss), 1e-12)
        h2_ref[0] = _pack_bf16(p_ref[0], RB)
        h2_ref[1] = _pack_bf16(p_ref[1], RB)
        hn_ref[...] = _pack_bf16(hn, RB)
        hnb_ref[...] = _pack_bf16(hn * beta_ref[0, 0], RB)

    norm = pl.pallas_call(
        _norm_body,
        grid=(GRID,),
        in_specs=[
            pl.BlockSpec((2, RB, Dh), lambda i: (0, i, 0)),
            pl.BlockSpec((1, 1), lambda i: (0, 0)),
        ],
        out_specs=[
            pl.BlockSpec((2, RB, Dhp), lambda i: (0, i, 0)),
            pl.BlockSpec((RB, Dp), lambda i: (i, 0)),
            pl.BlockSpec((RB, Dp), lambda i: (i, 0)),
        ],
        out_shape=[
            jax.ShapeDtypeStruct((2, NP, Dhp), f32),
            jax.ShapeDtypeStruct((N, Dp), f32),
            jax.ShapeDtypeStruct((N, Dp), f32),
        ],
    )

    def _cls_body(p_ref, w_ref, b_ref, y_ref):
        h = jnp.concatenate([p_ref[0], p_ref[1]], axis=1)
        y_ref[...] = jnp.dot(h, w_ref[...], preferred_element_type=f32) + b_ref[...]

    cls = pl.pallas_call(
        _cls_body,
        grid=(GRID,),
        in_specs=[
            pl.BlockSpec((2, RB, Dh), lambda i: (0, i, 0)),
            pl.BlockSpec((D, D), lambda i: (0, 0)),
            pl.BlockSpec((1, D), lambda i: (0, 0)),
        ],
        out_specs=pl.BlockSpec((RB, D), lambda i: (i, 0)),
        out_shape=jax.ShapeDtypeStruct((N, D), f32),
    )

    B = 80
    EPW = E // NW
    assert EPW * NW == E and EPW % B == 0 and B % _L == 0
    CH = EPW // B
    assert CH % 2 == 1
    EPT = E // NSUB
    CHB = EPT // B
    assert CHB % 2 == 0 and CHB >= 4
    mesh = plsc.VectorSubcoreMesh(core_axis_name="c", subcore_axis_name="s")
    sc_params = pltpu.CompilerParams(needs_layout_passes=False)
    sc_params_untiled = pltpu.CompilerParams(
        needs_layout_passes=False, use_tc_tiling_on_sc=False)

    @functools.partial(
        pl.kernel, mesh=mesh,
        out_type=(
            jax.ShapeDtypeStruct((NW, CH, B), f32),
            jax.ShapeDtypeStruct((2, NP), f32),
        ),
        scratch_types=[
            pltpu.VMEM((CH, B), i32),
            pltpu.VMEM((CH, B), i32),
            pltpu.VMEM((CH, B), f32),
            pltpu.VMEM((2, B, Dp), f32),
            pltpu.VMEM((2, B, Dp), f32),
            pltpu.VMEM_SHARED((NP,), f32),
            pltpu.SemaphoreType.DMA,
            pltpu.SemaphoreType.DMA,
            pltpu.SemaphoreType.DMA,
            pltpu.SemaphoreType.DMA,
        ],
        compiler_params=sc_params_untiled,
    )
    def pass_a(hn_hbm, hnb_hbm, src_hbm, dst_hbm, zvec_hbm, ez_hbm, den_hbm,
               idx_s, idx_d, ezv, rows_s, rows_d, den_sh, s0, s1, d0, d1):
        c = lax.axis_index("c")
        s = lax.axis_index("s")
        wid = c * NSUB + s
        pltpu.sync_copy(src_hbm.at[wid], idx_s)
        pltpu.sync_copy(dst_hbm.at[wid], idx_d)
        pltpu.sync_copy(zvec_hbm.at[pl.ds(s * STRIPE, STRIPE)],
                        den_sh.at[pl.ds(s * STRIPE, STRIPE)])
        plsc.subcore_barrier()

        sems = ((s0, d0), (s1, d1))

        def issue(ci, buf):
            ss_, dd_ = sems[buf]
            pltpu.async_copy(hnb_hbm.at[idx_s.at[ci]], rows_s.at[buf], ss_)
            pltpu.async_copy(hn_hbm.at[idx_d.at[ci]], rows_d.at[buf], dd_)

        def wait(buf):
            ss_, dd_ = sems[buf]
            pltpu.make_async_copy(hnb_hbm.at[idx_s.at[0]],
                                  rows_s.at[buf], ss_).wait()
            pltpu.make_async_copy(hn_hbm.at[idx_d.at[0]],
                                  rows_d.at[buf], dd_).wait()

        lane = lax.iota(i32, _L)
        perms = [lane ^ sh for sh in (8, 4, 2, 1)]
        himask = jnp.full((_L,), 0xFFFF0000, jnp.uint32)

        def unpk(w):
            u = lax.bitcast_convert_type(w, jnp.uint32)
            hi = lax.bitcast_convert_type(u & himask, f32)
            lo = lax.bitcast_convert_type(u << 16, f32)
            return lo, hi

        def compute(ci, buf):
            def grpf(g, cc):
                tot = jnp.zeros((_L,), f32)
                for j in range(_L):
                    e = g * _L + j
                    acc = jnp.zeros((_L,), f32)
                    for k in range(Dp // _L):
                        ls, hs = unpk(rows_s[buf, e, pl.ds(k * _L, _L)])
                        ld_, hd = unpk(rows_d[buf, e, pl.ds(k * _L, _L)])
                        acc = acc + ls * ld_
                        acc = acc + hs * hd
                    for p in perms:
                        acc = acc + acc.at[p].get(mode="promise_in_bounds")
                    tot = jnp.where(lane == j, acc, tot)
                ezv[ci, pl.ds(g * _L, _L)] = jnp.exp(tot)
                return cc

            lax.fori_loop(0, B // _L, grpf, 0)
            pltpu.sync_copy(ezv.at[ci], den_sh.at[idx_d.at[ci]], add=True)

        issue(0, 0)

        def body(g, cc):
            c0 = 2 * g
            issue(c0 + 1, 1)
            wait(0)
            compute(c0, 0)
            issue(c0 + 2, 0)
            wait(1)
            compute(c0 + 1, 1)
            return cc

        lax.fori_loop(0, (CH - 1) // 2, body, 0)
        wait(0)
        compute(CH - 1, 0)

        pltpu.sync_copy(ezv, ez_hbm.at[wid])
        plsc.subcore_barrier()
        pltpu.sync_copy(den_sh.at[pl.ds(s * STRIPE, STRIPE)],
                        den_hbm.at[c, pl.ds(s * STRIPE, STRIPE)])

    @functools.partial(
        pl.kernel, mesh=mesh,
        out_type=jax.ShapeDtypeStruct((2, NP, Dh), f32),
        scratch_types=[
            pltpu.VMEM((CHB, B), i32),
            pltpu.VMEM((CHB, B), i32),
            pltpu.VMEM((2, B), f32),
            pltpu.VMEM((2, B, Dhp), f32),
            pltpu.VMEM((2, B, Dh), f32),
            pltpu.VMEM((NP,), f32),
            pltpu.VMEM((NP,), f32),
            pltpu.VMEM_SHARED((NP, Dh), f32),
            pltpu.SemaphoreType.DMA,
            pltpu.SemaphoreType.DMA,
            pltpu.SemaphoreType.DMA,
            pltpu.SemaphoreType.DMA,
            pltpu.SemaphoreType.DMA,
            pltpu.SemaphoreType.DMA,
        ],
        compiler_params=sc_params_untiled,
    )
    def pass_b(h2_hbm, ez_hbm, src_hbm, dst_hbm, den_hbm, zmat_hbm, out_hbm,
               idx_s, idx_d, ezb, rows, wrow, den_v, den_v2, out_sh,
               g0, g1, e0, e1, w0, w1):
        c = lax.axis_index("c")
        s = lax.axis_index("s")
        ht = h2_hbm.at[c]
        pltpu.sync_copy(src_hbm.at[s], idx_s)
        pltpu.sync_copy(dst_hbm.at[s], idx_d)
        pltpu.sync_copy(zmat_hbm.at[pl.ds(s * STRIPE, STRIPE)],
                        out_sh.at[pl.ds(s * STRIPE, STRIPE)])
        pltpu.sync_copy(den_hbm.at[0], den_v)
        pltpu.sync_copy(den_hbm.at[1], den_v2)

        def dsum(i, cc):
            den_v[pl.ds(i * _L, _L)] = (den_v[pl.ds(i * _L, _L)]
                                        + den_v2[pl.ds(i * _L, _L)])
            return cc

        lax.fori_loop(0, NP // _L, dsum, 0)
        plsc.subcore_barrier()

        gsems = (g0, g1)
        esems = (e0, e1)
        wsems = (w0, w1)

        himask = jnp.full((_L,), 0xFFFF0000, jnp.uint32)

        def unpk(w):
            u = lax.bitcast_convert_type(w, jnp.uint32)
            hi = lax.bitcast_convert_type(u & himask, f32)
            lo = lax.bitcast_convert_type(u << 16, f32)
            return lo, hi

        def issue(ci, buf):
            pltpu.async_copy(ht.at[idx_s.at[ci]], rows.at[buf], gsems[buf])
            pltpu.async_copy(ez_hbm.at[s, ci], ezb.at[buf], esems[buf])

        def wait_g(buf):
            pltpu.make_async_copy(ht.at[idx_s.at[0]],
                                  rows.at[buf], gsems[buf]).wait()
            pltpu.make_async_copy(ez_hbm.at[s, 0],
                                  ezb.at[buf], esems[buf]).wait()

        def wait_w(buf):
            pltpu.make_async_copy(wrow.at[buf],
                                  out_sh.at[idx_d.at[0]], wsems[buf]).wait()

        def compute(ci, buf, prev_scatter):
            wait_g(buf)
            if prev_scatter is None:
                wait_w(buf)
            else:
                @pl.when(prev_scatter)
                def _():
                    wait_w(buf)

            def grpf(g2, cc):
                d16 = idx_d[ci, pl.ds(g2 * _L, _L)]
                den = plsc.load_gather(den_v, [d16])
                a16 = ezb[buf, pl.ds(g2 * _L, _L)] / den
                for j in range(_L):
                    av = a16.at[jnp.full((_L,), j, i32)].get(
                        mode="promise_in_bounds")
                    e = g2 * _L + j
                    for k in range(Dhp // _L):
                        lo, hi = unpk(rows[buf, e, pl.ds(k * _L, _L)])
                        wrow[buf, e, pl.ds(k * _L, _L)] = lo * av
                        wrow[buf, e, pl.ds(Dhp + k * _L, _L)] = hi * av
                return cc

            lax.fori_loop(0, B // _L, grpf, 0)
            pltpu.async_copy(wrow.at[buf], out_sh.at[idx_d.at[ci]],
                             wsems[buf], add=True)

        issue(0, 0)

        def body(g, cc):
            c0 = 2 * g
            issue(c0 + 1, 1)
            compute(c0, 0, g > 0)
            issue(c0 + 2, 0)
            compute(c0 + 1, 1, g > 0)
            return cc

        lax.fori_loop(0, CHB // 2 - 1, body, 0)
        issue(CHB - 1, 1)
        compute(CHB - 2, 0, None)
        compute(CHB - 1, 1, None)
        wait_w(0)
        wait_w(1)

        plsc.subcore_barrier()
        pltpu.sync_copy(out_sh.at[pl.ds(s * STRIPE, STRIPE)],
                        out_hbm.at[c, pl.ds(s * STRIPE, STRIPE)])

    return proj, norm, cls, pass_a, pass_b, NW, NSUB, CH, CHB, B


def kernel(features, edge_index, W1, b1, betas, Wc, bc):
    N, D = features.shape
    E = edge_index.shape[1]
    proj, norm, cls, pass_a, pass_b, NW, NSUB, CH, CHB, B = _build(N, E, D)
    NP = _round_up(N, NSUB * _L)

    src_a = edge_index[0].reshape(NW, CH, B)
    dst_a = edge_index[1].reshape(NW, CH, B)
    src_b = edge_index[0].reshape(NSUB, CHB, B)
    dst_b = edge_index[1].reshape(NSUB, CHB, B)
    zvec = jnp.zeros((NP,), jnp.float32)
    zmat = jnp.zeros((NP, D // 2), jnp.float32)

    h2p, hnp, hnbp = proj(features, W1, b1.reshape(1, D),
                          betas[0].reshape(1, 1))
    for i in range(betas.shape[0]):
        ez, den = pass_a(hnp, hnbp, src_a, dst_a, zvec)
        ezb = ez.reshape(NSUB, CHB, B)
        part = pass_b(h2p, ezb, src_b, dst_b, den, zmat)
        if i + 1 < betas.shape[0]:
            h2p, hnp, hnbp = norm(part, betas[i + 1].reshape(1, 1))
    return cls(part, Wc, bc.reshape(1, D))
